# baseline (speedup 1.0000x reference)
import functools

import jax
import jax.numpy as jnp
from jax import lax
from jax.experimental import pallas as pl
from jax.experimental.pallas import tpu as pltpu
from jax.experimental.pallas import tpu_sc as plsc

EMB_DIM = 64
POS_DIM = 25
ENTITY_DIM = 128
N_POS = 9

NUM_CORES = 2
NUM_SUBCORES = 16
NUM_WORKERS = NUM_CORES * NUM_SUBCORES


def _sc_gather(ids_half, flat_ids, table_pairs, entity_id_to_pos_index):
    n = flat_ids.shape[0]
    per_w = n // NUM_WORKERS
    chunk = 400
    n_chunks = per_w // chunk
    assert per_w % chunk == 0 and per_w * NUM_WORKERS == n

    mesh = plsc.VectorSubcoreMesh(core_axis_name="c", subcore_axis_name="s")

    @functools.partial(
        pl.kernel,
        mesh=mesh,
        out_type=[
            jax.ShapeDtypeStruct((n, ENTITY_DIM), jnp.float32),
            jax.ShapeDtypeStruct((n,), jnp.int32),
        ],
        scratch_types=[
            pltpu.VMEM((chunk,), jnp.int32),
            pltpu.VMEM((chunk,), jnp.int32),
            pltpu.VMEM((chunk, ENTITY_DIM), jnp.float32),
            pltpu.VMEM((chunk,), jnp.int32),
            pltpu.SemaphoreType.DMA,
            pltpu.SemaphoreType.DMA,
        ],
    )
    def k(idsh_hbm, ids_hbm, table_hbm, eip_hbm, pair_out, pidx_out,
          idxh_v, idx_v, rows_v, pidx_v, sem_rows, sem_pidx):
        wid = lax.axis_index("s") * NUM_CORES + lax.axis_index("c")
        for ci in range(n_chunks):
            base = wid * per_w + ci * chunk
            pltpu.sync_copy(idsh_hbm.at[pl.ds(base, chunk)], idxh_v)
            pltpu.sync_copy(ids_hbm.at[pl.ds(base, chunk)], idx_v)
            cp_rows = pltpu.async_copy(table_hbm.at[idxh_v], rows_v, sem_rows)
            cp_pidx = pltpu.async_copy(eip_hbm.at[idx_v], pidx_v, sem_pidx)
            cp_rows.wait()
            cp_pidx.wait()
            pltpu.sync_copy(rows_v, pair_out.at[pl.ds(base, chunk)])
            pltpu.sync_copy(pidx_v, pidx_out.at[pl.ds(base, chunk)])

    return k(ids_half, flat_ids, table_pairs, entity_id_to_pos_index)


def _tc_project(pair_rows, pidx, parity, we_t2, pos_head, wp_t, bias):
    n = pair_rows.shape[0]
    blk = 2048
    grid = n // blk
    assert n % blk == 0

    def body(pair_ref, pidx_ref, par_ref, wet2_ref, ph_ref, wpt_ref, b_ref,
             out_ref):
        pp = jnp.dot(ph_ref[...], wpt_ref[...],
                     preferred_element_type=jnp.float32)
        iota = lax.broadcasted_iota(jnp.int32, (blk, ENTITY_DIM), 1)
        oh = (pidx_ref[...] == iota).astype(jnp.float32)
        in_hi = iota >= EMB_DIM
        keep = (par_ref[...] == in_hi.astype(jnp.int32)).astype(jnp.float32)
        ent2 = pair_ref[...] * keep
        out_ref[...] = (
            jnp.dot(ent2, wet2_ref[...], preferred_element_type=jnp.float32)
            + jnp.dot(oh, pp, preferred_element_type=jnp.float32)
            + b_ref[...]
        )

    return pl.pallas_call(
        body,
        grid=(grid,),
        in_specs=[
            pl.BlockSpec((blk, ENTITY_DIM), lambda i: (i, 0)),
            pl.BlockSpec((blk, 1), lambda i: (i, 0)),
            pl.BlockSpec((blk, 1), lambda i: (i, 0)),
            pl.BlockSpec((ENTITY_DIM, ENTITY_DIM), lambda i: (0, 0)),
            pl.BlockSpec((ENTITY_DIM, 32), lambda i: (0, 0)),
            pl.BlockSpec((32, ENTITY_DIM), lambda i: (0, 0)),
            pl.BlockSpec((1, ENTITY_DIM), lambda i: (0, 0)),
        ],
        out_specs=pl.BlockSpec((blk, ENTITY_DIM), lambda i: (i, 0)),
        out_shape=jax.ShapeDtypeStruct((n, ENTITY_DIM), jnp.float32),
    )(pair_rows, pidx, parity, we_t2, pos_head, wp_t, bias)


def kernel(entity_ids, entity_table, entity_id_to_pos_index, pos_table, W, b):
    shape = entity_ids.shape
    flat_ids = entity_ids.reshape(-1)
    ids_half = flat_ids >> 1
    parity = (flat_ids & 1).reshape(-1, 1)
    table_pairs = entity_table.reshape(-1, 2 * EMB_DIM)

    pair_rows, pidx = _sc_gather(ids_half, flat_ids, table_pairs,
                                 entity_id_to_pos_index)

    we_t = W[:, :EMB_DIM].T
    we_t2 = jnp.concatenate([we_t, we_t], axis=0)
    wp_t = jnp.zeros((32, ENTITY_DIM), jnp.float32).at[:POS_DIM].set(
        W[:, EMB_DIM:].T)
    pos_head = jnp.zeros((ENTITY_DIM, 32), jnp.float32).at[:N_POS, :POS_DIM].set(
        pos_table[:N_POS])
    bias = b.reshape(1, ENTITY_DIM)

    out = _tc_project(pair_rows, pidx.reshape(-1, 1), parity,
                      we_t2, pos_head, wp_t, bias)
    return out.reshape(*shape, ENTITY_DIM)


# speedup vs baseline: 1.0544x; 1.0544x over previous
import functools

import jax
import jax.numpy as jnp
from jax import lax
from jax.experimental import pallas as pl
from jax.experimental.pallas import tpu as pltpu
from jax.experimental.pallas import tpu_sc as plsc

EMB_DIM = 64
POS_DIM = 25
ENTITY_DIM = 128
N_POS = 9

NUM_CORES = 2
NUM_SUBCORES = 16
NUM_WORKERS = NUM_CORES * NUM_SUBCORES


def _sc_gather(ids_half, flat_ids, table_flat, entity_id_to_pos_index):
    n = flat_ids.shape[0]
    per_w = n // NUM_WORKERS
    chunk = 400
    n_chunks = per_w // chunk
    mesh = plsc.VectorSubcoreMesh(core_axis_name="c", subcore_axis_name="s")

    @functools.partial(
        pl.kernel,
        mesh=mesh,
        out_type=[
            jax.ShapeDtypeStruct((n, ENTITY_DIM), jnp.float32),
            jax.ShapeDtypeStruct((n,), jnp.int32),
        ],
        scratch_types=[
            pltpu.VMEM((chunk,), jnp.int32),
            pltpu.VMEM((chunk,), jnp.int32),
            pltpu.VMEM((chunk, ENTITY_DIM), jnp.float32),
            pltpu.VMEM((chunk,), jnp.int32),
            pltpu.SemaphoreType.DMA,
            pltpu.SemaphoreType.DMA,
        ],
    )
    def k(idsh_hbm, ids_hbm, table_pairs, eip_hbm, pair_out, pidx_out,
          idxh_v, idx_v, rows_v, pidx_v, sem_rows, sem_pidx):
        wid = lax.axis_index("s") * NUM_CORES + lax.axis_index("c")
        for ci in range(n_chunks):
            base = wid * per_w + ci * chunk
            pltpu.sync_copy(idsh_hbm.at[pl.ds(base, chunk)], idxh_v)
            pltpu.sync_copy(ids_hbm.at[pl.ds(base, chunk)], idx_v)
            cp_rows = pltpu.async_copy(table_pairs.at[idxh_v], rows_v, sem_rows)
            cp_pidx = pltpu.async_copy(eip_hbm.at[idx_v], pidx_v, sem_pidx)
            cp_rows.wait()
            cp_pidx.wait()
            pltpu.sync_copy(rows_v, pair_out.at[pl.ds(base, chunk)])
            pltpu.sync_copy(pidx_v, pidx_out.at[pl.ds(base, chunk)])

    return k(ids_half, flat_ids, table_flat, entity_id_to_pos_index)


def _tc_project(pair_rows, pidx, parity, we_t2, pos_head, wp_t, bias, shape):
    n = pair_rows.shape[0]
    d0, d1, d2 = shape
    blk0 = 32
    blk = blk0 * d1 * d2
    grid = d0 // blk0
    assert n % blk == 0 and d0 % blk0 == 0

    def body(pair_ref, pidx_ref, par_ref, wet2_ref, ph_ref, wpt_ref, b_ref,
             out_ref):
        pp = jnp.dot(ph_ref[...], wpt_ref[...],
                     preferred_element_type=jnp.float32)
        iota = lax.broadcasted_iota(jnp.int32, (blk, ENTITY_DIM), 1)
        oh = (pidx_ref[...] == iota).astype(jnp.float32)
        in_hi = iota >= EMB_DIM
        keep = (par_ref[...] == in_hi.astype(jnp.int32)).astype(jnp.float32)
        ent2 = pair_ref[...] * keep
        out_ref[...] = (
            jnp.dot(ent2, wet2_ref[...], preferred_element_type=jnp.float32)
            + jnp.dot(oh, pp, preferred_element_type=jnp.float32)
            + b_ref[...]
        ).reshape(blk0, d1, d2, ENTITY_DIM)

    return pl.pallas_call(
        body,
        grid=(grid,),
        in_specs=[
            pl.BlockSpec((blk, ENTITY_DIM), lambda i: (i, 0)),
            pl.BlockSpec((blk, 1), lambda i: (i, 0)),
            pl.BlockSpec((blk, 1), lambda i: (i, 0)),
            pl.BlockSpec((ENTITY_DIM, ENTITY_DIM), lambda i: (0, 0)),
            pl.BlockSpec((ENTITY_DIM, 32), lambda i: (0, 0)),
            pl.BlockSpec((32, ENTITY_DIM), lambda i: (0, 0)),
            pl.BlockSpec((1, ENTITY_DIM), lambda i: (0, 0)),
        ],
        out_specs=pl.BlockSpec((blk0, d1, d2, ENTITY_DIM),
                               lambda i: (i, 0, 0, 0)),
        out_shape=jax.ShapeDtypeStruct((d0, d1, d2, ENTITY_DIM), jnp.float32),
    )(pair_rows, pidx, parity, we_t2, pos_head, wp_t, bias)


def kernel(entity_ids, entity_table, entity_id_to_pos_index, pos_table, W, b):
    shape = entity_ids.shape
    flat_ids = entity_ids.reshape(-1)
    ids_half = flat_ids >> 1
    parity = (flat_ids & 1).reshape(-1, 1)
    table_flat = entity_table.reshape(-1, ENTITY_DIM)

    pair_rows, pidx = _sc_gather(ids_half, flat_ids, table_flat,
                                 entity_id_to_pos_index)

    we_t = W[:, :EMB_DIM].T
    we_t2 = jnp.concatenate([we_t, we_t], axis=0)
    wp_t = jnp.zeros((32, ENTITY_DIM), jnp.float32).at[:POS_DIM].set(
        W[:, EMB_DIM:].T)
    pos_head = jnp.zeros((ENTITY_DIM, 32), jnp.float32).at[:N_POS, :POS_DIM].set(
        pos_table[:N_POS])
    bias = b.reshape(1, ENTITY_DIM)

    return _tc_project(pair_rows, pidx.reshape(-1, 1), parity,
                       we_t2, pos_head, wp_t, bias, shape)


# speedup vs baseline: 1.0568x; 1.0023x over previous
import functools

import jax
import jax.numpy as jnp
from jax import lax
from jax.experimental import pallas as pl
from jax.experimental.pallas import tpu as pltpu
from jax.experimental.pallas import tpu_sc as plsc

EMB_DIM = 64
POS_DIM = 25
ENTITY_DIM = 128
N_POS = 9

NUM_CORES = 2
NUM_SUBCORES = 16
NUM_WORKERS = NUM_CORES * NUM_SUBCORES


def _sc_gather(ids_half, flat_ids, table_flat, entity_id_to_pos_index):
    n = flat_ids.shape[0]
    per_w = n // NUM_WORKERS
    chunk = 800
    n_chunks = per_w // chunk
    mesh = plsc.VectorSubcoreMesh(core_axis_name="c", subcore_axis_name="s")

    @functools.partial(
        pl.kernel,
        mesh=mesh,
        out_type=[
            jax.ShapeDtypeStruct((n, ENTITY_DIM), jnp.float32),
            jax.ShapeDtypeStruct((n,), jnp.int32),
        ],
        scratch_types=[
            pltpu.VMEM((chunk,), jnp.int32),
            pltpu.VMEM((chunk,), jnp.int32),
            pltpu.VMEM((chunk, ENTITY_DIM), jnp.float32),
            pltpu.VMEM((chunk,), jnp.int32),
            pltpu.SemaphoreType.DMA,
            pltpu.SemaphoreType.DMA,
        ],
    )
    def k(idsh_hbm, ids_hbm, table_pairs, eip_hbm, pair_out, pidx_out,
          idxh_v, idx_v, rows_v, pidx_v, sem_rows, sem_pidx):
        wid = lax.axis_index("s") * NUM_CORES + lax.axis_index("c")
        for ci in range(n_chunks):
            base = wid * per_w + ci * chunk
            pltpu.sync_copy(idsh_hbm.at[pl.ds(base, chunk)], idxh_v)
            pltpu.sync_copy(ids_hbm.at[pl.ds(base, chunk)], idx_v)
            cp_rows = pltpu.async_copy(table_pairs.at[idxh_v], rows_v, sem_rows)
            cp_pidx = pltpu.async_copy(eip_hbm.at[idx_v], pidx_v, sem_pidx)
            cp_rows.wait()
            cp_pidx.wait()
            pltpu.sync_copy(rows_v, pair_out.at[pl.ds(base, chunk)])
            pltpu.sync_copy(pidx_v, pidx_out.at[pl.ds(base, chunk)])

    return k(ids_half, flat_ids, table_flat, entity_id_to_pos_index)


def _tc_project(pair_rows, pidx, parity, we_t2, pos_head, wp_t, bias, shape):
    n = pair_rows.shape[0]
    d0, d1, d2 = shape
    blk0 = 32
    blk = blk0 * d1 * d2
    grid = d0 // blk0
    assert n % blk == 0 and d0 % blk0 == 0

    def body(pair_ref, pidx_ref, par_ref, wet2_ref, ph_ref, wpt_ref, b_ref,
             out_ref):
        pp = jnp.dot(ph_ref[...], wpt_ref[...],
                     preferred_element_type=jnp.float32)
        iota = lax.broadcasted_iota(jnp.int32, (blk, ENTITY_DIM), 1)
        oh = (pidx_ref[...] == iota).astype(jnp.float32)
        in_hi = iota >= EMB_DIM
        keep = (par_ref[...] == in_hi.astype(jnp.int32)).astype(jnp.float32)
        ent2 = pair_ref[...] * keep
        out_ref[...] = (
            jnp.dot(ent2, wet2_ref[...], preferred_element_type=jnp.float32)
            + jnp.dot(oh, pp, preferred_element_type=jnp.float32)
            + b_ref[...]
        ).reshape(blk0, d1, d2, ENTITY_DIM)

    return pl.pallas_call(
        body,
        grid=(grid,),
        in_specs=[
            pl.BlockSpec((blk, ENTITY_DIM), lambda i: (i, 0)),
            pl.BlockSpec((blk, 1), lambda i: (i, 0)),
            pl.BlockSpec((blk, 1), lambda i: (i, 0)),
            pl.BlockSpec((ENTITY_DIM, ENTITY_DIM), lambda i: (0, 0)),
            pl.BlockSpec((ENTITY_DIM, 32), lambda i: (0, 0)),
            pl.BlockSpec((32, ENTITY_DIM), lambda i: (0, 0)),
            pl.BlockSpec((1, ENTITY_DIM), lambda i: (0, 0)),
        ],
        out_specs=pl.BlockSpec((blk0, d1, d2, ENTITY_DIM),
                               lambda i: (i, 0, 0, 0)),
        out_shape=jax.ShapeDtypeStruct((d0, d1, d2, ENTITY_DIM), jnp.float32),
    )(pair_rows, pidx, parity, we_t2, pos_head, wp_t, bias)


def kernel(entity_ids, entity_table, entity_id_to_pos_index, pos_table, W, b):
    shape = entity_ids.shape
    flat_ids = entity_ids.reshape(-1)
    ids_half = flat_ids >> 1
    parity = (flat_ids & 1).reshape(-1, 1)
    table_flat = entity_table.reshape(-1, ENTITY_DIM)

    pair_rows, pidx = _sc_gather(ids_half, flat_ids, table_flat,
                                 entity_id_to_pos_index)

    we_t = W[:, :EMB_DIM].T
    we_t2 = jnp.concatenate([we_t, we_t], axis=0)
    wp_t = jnp.zeros((32, ENTITY_DIM), jnp.float32).at[:POS_DIM].set(
        W[:, EMB_DIM:].T)
    pos_head = jnp.zeros((ENTITY_DIM, 32), jnp.float32).at[:N_POS, :POS_DIM].set(
        pos_table[:N_POS])
    bias = b.reshape(1, ENTITY_DIM)

    return _tc_project(pair_rows, pidx.reshape(-1, 1), parity,
                       we_t2, pos_head, wp_t, bias, shape)
